# baseline (speedup 1.0000x reference)
import functools

import jax
import jax.numpy as jnp
from jax import lax
from jax.experimental import pallas as pl
from jax.experimental.pallas import tpu as pltpu
from jax.experimental.pallas import tpu_sc as plsc


def _elu(x):
    return jnp.where(x > 0, x, jnp.exp(x) - 1.0)


def _bdot(a, b):
    return jnp.dot(a.astype(jnp.bfloat16), b.astype(jnp.bfloat16),
                   preferred_element_type=jnp.float32)


def _attr_kernel(x_ref, a_ref, wpre_ref, bpre_ref, wall_ref, bops_ref,
                 wres1_ref, bres1_ref, wres2_ref, bres2_ref, out_ref, *, n_ops):
    h = _bdot(x_ref[...], wpre_ref[...]) + bpre_ref[...]
    a = a_ref[0]
    d = h.shape[1]
    big = _bdot(h, wall_ref[...])
    ks = 1 + jax.lax.broadcasted_iota(jnp.int32, (1, n_ops), 1)
    oh = (a == ks).astype(jnp.float32)
    acc = jnp.dot(oh, bops_ref[...], preferred_element_type=jnp.float32)
    for k in range(1, n_ops + 1):
        acc = acc + jnp.where(a == k, big[:, (k - 1) * d:k * d], 0.0)
    acc = _elu(acc)
    acc = jnp.where(a == 0, 0.0, acc)
    r = _elu(_bdot(acc, wres1_ref[...]) + bres1_ref[...])
    r = _elu(_bdot(r, wres2_ref[...]) + bres2_ref[...])
    out_ref[...] = _elu(acc + r) + h


def _compact_kernel(aT_ref, idx_ref, cidx_ref, caval_ref, cnt_ref, *,
                    bc, sent):
    aT = aT_ref[0]
    mT = (aT == 0)
    mTf = mT.astype(jnp.float32)
    mCf = 1.0 - mTf
    iota_ri = lax.broadcasted_iota(jnp.int32, (bc, bc), 0)
    iota_ci = lax.broadcasted_iota(jnp.int32, (bc, bc), 1)
    iota_r = iota_ri.astype(jnp.float32)
    tri = (iota_ri <= iota_ci).astype(jnp.float32)
    gidx = (bc * pl.program_id(0)
            + lax.broadcasted_iota(jnp.int32, (1, bc), 1)).astype(jnp.float32)
    slotT = lax.broadcasted_iota(jnp.int32, (1, bc), 1).astype(jnp.float32)

    prefT = jnp.floor(
        jnp.dot(mTf, tri, preferred_element_type=jnp.float32) + 0.5)
    destT = prefT - 1.0
    cnt = jnp.sum(mTf)
    sel = jnp.where(mT, (iota_r == destT).astype(jnp.float32), 0.0)
    compactT = lax.dot_general(gidx, sel, (((1,), (1,)), ((), ())),
                               precision=lax.Precision.HIGHEST,
                               preferred_element_type=jnp.float32)
    compactT = compactT + jnp.where(slotT >= cnt, float(sent), 0.0)
    idx_ref[0] = jnp.floor(compactT + 0.5).astype(jnp.int32)
    cnt_ref[0] = jnp.full((1, 16), cnt).astype(jnp.int32)

    prefC = jnp.floor(
        jnp.dot(mCf, tri, preferred_element_type=jnp.float32) + 0.5)
    destC = prefC - 1.0
    cntC = bc - cnt
    selC = jnp.where(aT != 0, (iota_r == destC).astype(jnp.float32), 0.0)
    cidxT = lax.dot_general(gidx, selC, (((1,), (1,)), ((), ())),
                            precision=lax.Precision.HIGHEST,
                            preferred_element_type=jnp.float32)
    cidxT = cidxT + jnp.where(slotT >= cntC, float(sent), 0.0)
    cidx_ref[0] = jnp.floor(cidxT + 0.5).astype(jnp.int32)
    cavalT = lax.dot_general(aT.astype(jnp.float32), selC,
                             (((1,), (1,)), ((), ())),
                             precision=lax.Precision.HIGHEST,
                             preferred_element_type=jnp.float32)
    caval_ref[0] = jnp.floor(cavalT + 0.5).astype(jnp.int32)


def _sc_gather(idx_hbm, cnt_hbm, emb_hbm, gat_out,
               idxt_v, cnt_v, rows_v, sem, *, rcap, gtile, ncores, rper):
    wid = lax.axis_index("s") * ncores + lax.axis_index("c")
    for rr in range(rper):
        region = wid * rper + rr
        rbase = region * rcap
        pltpu.sync_copy(cnt_hbm.at[region], cnt_v)
        tot = jnp.max(cnt_v[...])
        ntiles = lax.div(tot + gtile - 1, gtile)

        def gbody(t, _):
            @pl.when(t < ntiles)
            def _():
                pltpu.sync_copy(idx_hbm.at[pl.ds(rbase + t * gtile, gtile)],
                                idxt_v)
                pltpu.async_copy(emb_hbm.at[idxt_v], rows_v, sem).wait()
                pltpu.sync_copy(rows_v,
                                gat_out.at[pl.ds(rbase + t * gtile, gtile)])
            return 0

        lax.fori_loop(0, rcap // gtile, gbody, 0, unroll=False)


def _res_mlp_kernel(cnt_ref, gat_ref, embb_ref, wres1_ref, bres1_ref,
                    wres2_ref, bres2_ref, outc_ref, x_v, o_v, sem_in, sem_out,
                    *, rcap, tb):
    r = pl.program_id(0)
    cnt = cnt_ref[r]
    ntiles = (cnt + tb - 1) // tb

    def body(t, _):
        start = r * rcap + t * tb
        cp_in = pltpu.make_async_copy(gat_ref.at[pl.ds(start, tb)], x_v,
                                      sem_in)
        cp_in.start()
        cp_in.wait()
        h = x_v[...] + embb_ref[...]
        z = _elu(_bdot(h, wres1_ref[...]) + bres1_ref[...])
        z = _elu(_bdot(z, wres2_ref[...]) + bres2_ref[...])
        o_v[...] = _elu(h + z)
        cp_out = pltpu.make_async_copy(o_v, outc_ref.at[pl.ds(start, tb)],
                                       sem_out)
        cp_out.start()
        cp_out.wait()
        return 0

    lax.fori_loop(0, ntiles, body, 0, unroll=False)


def _dtable_kernel(bops_ref, wres1_ref, bres1_ref, wres2_ref, bres2_ref,
                   out_ref):
    c = _elu(bops_ref[...])
    z = _elu(jnp.dot(c, wres1_ref[...],
                     preferred_element_type=jnp.float32) + bres1_ref[...])
    z = _elu(jnp.dot(z, wres2_ref[...],
                     preferred_element_type=jnp.float32) + bres2_ref[...])
    out_ref[...] = _elu(c + z)


def _sc_write(dtab_hbm, outc_hbm, idx_hbm, cidx_hbm, caval_hbm, cnt_hbm,
              out_hbm, at_v, idxt_v, cnt_v, rows_v, rows2_v, sem, sem2,
              *, rcap, gtile, ncores, rper):
    wid = lax.axis_index("s") * ncores + lax.axis_index("c")
    for rr in range(rper):
        region = wid * rper + rr
        rbase = region * rcap
        pltpu.sync_copy(cnt_hbm.at[region], cnt_v)
        tot = jnp.max(cnt_v[...])
        ntB = lax.div(tot + gtile - 1, gtile)
        ntA = lax.div((rcap - tot) + gtile - 1, gtile)

        def abody(t, _):
            @pl.when(t < ntA)
            def _():
                pltpu.sync_copy(caval_hbm.at[pl.ds(rbase + t * gtile, gtile)],
                                at_v)
                pltpu.async_copy(dtab_hbm.at[at_v], rows_v, sem).wait()
                pltpu.sync_copy(cidx_hbm.at[pl.ds(rbase + t * gtile, gtile)],
                                idxt_v)
                pltpu.async_copy(rows_v, out_hbm.at[idxt_v], sem2).wait()
            return 0

        lax.fori_loop(0, rcap // gtile, abody, 0, unroll=False)

        def bbody(t, _):
            @pl.when(t < ntB)
            def _():
                pltpu.sync_copy(outc_hbm.at[pl.ds(rbase + t * gtile, gtile)],
                                rows2_v)
                pltpu.sync_copy(idx_hbm.at[pl.ds(rbase + t * gtile, gtile)],
                                idxt_v)
                pltpu.async_copy(rows2_v, out_hbm.at[idxt_v], sem2).wait()
            return 0

        lax.fori_loop(0, rcap // gtile, bbody, 0, unroll=False)


def kernel(x_attr, node_assign, W_pre, b_pre, emb_W, emb_b, W_ops, b_ops,
           W_res1, b_res1, W_res2, b_res2):
    n_attr, d_in = x_attr.shape
    n_total = node_assign.shape[0]
    n_unattr = n_total - n_attr
    n_ops, d_hid, _ = W_ops.shape
    d_mid = W_res1.shape[1]

    assign = node_assign.astype(jnp.int32)

    info = plsc.get_sparse_core_info()
    ncores = info.num_cores
    nw = ncores * info.num_subcores

    B = 512
    BC = 640
    GT = 128
    RPER = 2
    pa = pl.cdiv(n_attr, B) * B
    pu = pl.cdiv(n_unattr, nw * BC * RPER) * nw * BC * RPER
    cap = pu // nw
    nregions = pu // BC
    sent = pu

    b_pre2 = b_pre.reshape(1, d_hid)
    emb_b2 = emb_b.reshape(1, d_hid)
    b_res1_2 = b_res1.reshape(1, d_mid)
    b_res2_2 = b_res2.reshape(1, d_hid)

    full = lambda shape: pl.BlockSpec(shape, lambda *_: (0,) * len(shape))

    x_p = jnp.pad(x_attr, ((0, pa - n_attr), (0, 0)))
    W_all = jnp.transpose(W_ops, (1, 0, 2)).reshape(d_hid, n_ops * d_hid)
    a_attr = jnp.pad(assign[:n_attr], (0, pa - n_attr)).reshape(pa // B, B, 1)
    out_attr = pl.pallas_call(
        functools.partial(_attr_kernel, n_ops=n_ops),
        grid=(pa // B,),
        in_specs=[
            pl.BlockSpec((B, d_in), lambda i: (i, 0)),
            pl.BlockSpec((1, B, 1), lambda i: (i, 0, 0)),
            full((d_in, d_hid)),
            full((1, d_hid)),
            full((d_hid, n_ops * d_hid)),
            full((n_ops, d_hid)),
            full((d_hid, d_mid)),
            full((1, d_mid)),
            full((d_mid, d_hid)),
            full((1, d_hid)),
        ],
        out_specs=pl.BlockSpec((B, d_hid), lambda i: (i, 0)),
        out_shape=jax.ShapeDtypeStruct((pa, d_hid), jnp.float32),
    )(x_p, a_attr, W_pre, b_pre2, W_all, b_ops, W_res1, b_res1_2,
      W_res2, b_res2_2)

    a_un = jnp.pad(assign[n_attr:], (0, pu - n_unattr), constant_values=1)
    aT3 = a_un.reshape(nregions, 1, BC)
    idx3, cidx3, caval3, cnt3 = pl.pallas_call(
        functools.partial(_compact_kernel, bc=BC, sent=sent),
        grid=(nregions,),
        in_specs=[pl.BlockSpec((1, 1, BC), lambda i: (i, 0, 0))],
        out_specs=[pl.BlockSpec((1, 1, BC), lambda i: (i, 0, 0)),
                   pl.BlockSpec((1, 1, BC), lambda i: (i, 0, 0)),
                   pl.BlockSpec((1, 1, BC), lambda i: (i, 0, 0)),
                   pl.BlockSpec((1, 1, 16), lambda i: (i, 0, 0))],
        out_shape=[jax.ShapeDtypeStruct((nregions, 1, BC), jnp.int32),
                   jax.ShapeDtypeStruct((nregions, 1, BC), jnp.int32),
                   jax.ShapeDtypeStruct((nregions, 1, BC), jnp.int32),
                   jax.ShapeDtypeStruct((nregions, 1, 16), jnp.int32)],
    )(aT3)
    idx_arr = idx3.reshape(pu)
    cidx_arr = cidx3.reshape(pu)
    caval_arr = caval3.reshape(pu)
    cnts = cnt3.reshape(nregions, 16)

    emb_p = jnp.pad(emb_W, ((0, pu + 8 - n_unattr), (0, 0)))
    mesh = plsc.VectorSubcoreMesh(core_axis_name="c", subcore_axis_name="s")
    s1 = pl.kernel(
        functools.partial(_sc_gather, rcap=BC, gtile=GT, ncores=ncores,
                          rper=RPER),
        out_type=jax.ShapeDtypeStruct((pu, d_hid), jnp.float32),
        mesh=mesh,
        compiler_params=pltpu.CompilerParams(needs_layout_passes=False),
        scratch_types=[
            pltpu.VMEM((GT,), jnp.int32),
            pltpu.VMEM((16,), jnp.int32),
            pltpu.VMEM((GT, d_hid), jnp.float32),
            pltpu.SemaphoreType.DMA,
        ],
    )
    gat = s1(idx_arr, cnts, emb_p)

    out_c = pl.pallas_call(
        functools.partial(_res_mlp_kernel, rcap=BC, tb=GT),
        grid_spec=pltpu.PrefetchScalarGridSpec(
            num_scalar_prefetch=1,
            grid=(nregions,),
            in_specs=[
                pl.BlockSpec(memory_space=pl.MemorySpace.ANY),
                full((1, d_hid)),
                full((d_hid, d_mid)),
                full((1, d_mid)),
                full((d_mid, d_hid)),
                full((1, d_hid)),
            ],
            out_specs=pl.BlockSpec(memory_space=pl.MemorySpace.ANY),
            scratch_shapes=[
                pltpu.VMEM((GT, d_hid), jnp.float32),
                pltpu.VMEM((GT, d_hid), jnp.float32),
                pltpu.SemaphoreType.DMA,
                pltpu.SemaphoreType.DMA,
            ],
        ),
        out_shape=jax.ShapeDtypeStruct((pu, d_hid), jnp.float32),
    )(cnts[:, 0], gat, emb_b2, W_res1, b_res1_2, W_res2, b_res2_2)

    bops_p = jnp.pad(b_ops, ((1, 0), (0, 0)))
    dtab = pl.pallas_call(
        _dtable_kernel,
        in_specs=[full((n_ops + 1, d_hid)), full((d_hid, d_mid)),
                  full((1, d_mid)), full((d_mid, d_hid)), full((1, d_hid))],
        out_specs=full((n_ops + 1, d_hid)),
        out_shape=jax.ShapeDtypeStruct((n_ops + 1, d_hid), jnp.float32),
    )(bops_p, W_res1, b_res1_2, W_res2, b_res2_2)

    s3 = pl.kernel(
        functools.partial(_sc_write, rcap=BC, gtile=GT,
                          ncores=ncores, rper=RPER),
        out_type=jax.ShapeDtypeStruct((pu + 8, d_hid), jnp.float32),
        mesh=mesh,
        compiler_params=pltpu.CompilerParams(needs_layout_passes=False),
        scratch_types=[
            pltpu.VMEM((GT,), jnp.int32),
            pltpu.VMEM((GT,), jnp.int32),
            pltpu.VMEM((16,), jnp.int32),
            pltpu.VMEM((GT, d_hid), jnp.float32),
            pltpu.VMEM((GT, d_hid), jnp.float32),
            pltpu.SemaphoreType.DMA,
            pltpu.SemaphoreType.DMA,
        ],
    )
    out_un = s3(dtab, out_c, idx_arr, cidx_arr, caval_arr, cnts)

    return jnp.concatenate([out_attr[:n_attr], out_un[:n_unattr]], axis=0)


# speedup vs baseline: 1.1487x; 1.1487x over previous
import functools

import jax
import jax.numpy as jnp
from jax import lax
from jax.experimental import pallas as pl
from jax.experimental.pallas import tpu as pltpu
from jax.experimental.pallas import tpu_sc as plsc


def _elu(x):
    return jnp.where(x > 0, x, jnp.exp(x) - 1.0)


def _bdot(a, b):
    return jnp.dot(a.astype(jnp.bfloat16), b.astype(jnp.bfloat16),
                   preferred_element_type=jnp.float32)


def _attr_kernel(x_ref, a_ref, wpre_ref, bpre_ref, wall_ref, bops_ref,
                 wres1_ref, bres1_ref, wres2_ref, bres2_ref, out_ref, *, n_ops):
    h = _bdot(x_ref[...], wpre_ref[...]) + bpre_ref[...]
    a = a_ref[0]
    d = h.shape[1]
    big = _bdot(h, wall_ref[...])
    ks = 1 + jax.lax.broadcasted_iota(jnp.int32, (1, n_ops), 1)
    oh = (a == ks).astype(jnp.float32)
    acc = jnp.dot(oh, bops_ref[...], preferred_element_type=jnp.float32)
    for k in range(1, n_ops + 1):
        acc = acc + jnp.where(a == k, big[:, (k - 1) * d:k * d], 0.0)
    acc = _elu(acc)
    acc = jnp.where(a == 0, 0.0, acc)
    r = _elu(_bdot(acc, wres1_ref[...]) + bres1_ref[...])
    r = _elu(_bdot(r, wres2_ref[...]) + bres2_ref[...])
    out_ref[...] = _elu(acc + r) + h


def _compact_kernel(aT_ref, idx_ref, cidx_ref, caval_ref, cnt_ref, *,
                    bc, sent):
    aT = aT_ref[0]
    mT = (aT == 0)
    mTf = mT.astype(jnp.float32)
    mCf = 1.0 - mTf
    iota_ri = lax.broadcasted_iota(jnp.int32, (bc, bc), 0)
    iota_ci = lax.broadcasted_iota(jnp.int32, (bc, bc), 1)
    iota_r = iota_ri.astype(jnp.float32)
    tri = (iota_ri <= iota_ci).astype(jnp.float32)
    gidx = (bc * pl.program_id(0)
            + lax.broadcasted_iota(jnp.int32, (1, bc), 1)).astype(jnp.float32)
    slotT = lax.broadcasted_iota(jnp.int32, (1, bc), 1).astype(jnp.float32)

    prefT = jnp.floor(
        jnp.dot(mTf, tri, preferred_element_type=jnp.float32) + 0.5)
    destT = prefT - 1.0
    cnt = jnp.sum(mTf)
    sel = jnp.where(mT, (iota_r == destT).astype(jnp.float32), 0.0)
    compactT = lax.dot_general(gidx, sel, (((1,), (1,)), ((), ())),
                               precision=lax.Precision.HIGHEST,
                               preferred_element_type=jnp.float32)
    compactT = compactT + jnp.where(slotT >= cnt, float(sent), 0.0)
    idx_ref[0] = jnp.floor(compactT + 0.5).astype(jnp.int32)
    cnt_ref[0] = jnp.full((1, 16), cnt).astype(jnp.int32)

    prefC = jnp.floor(
        jnp.dot(mCf, tri, preferred_element_type=jnp.float32) + 0.5)
    destC = prefC - 1.0
    cntC = bc - cnt
    selC = jnp.where(aT != 0, (iota_r == destC).astype(jnp.float32), 0.0)
    cidxT = lax.dot_general(gidx, selC, (((1,), (1,)), ((), ())),
                            precision=lax.Precision.HIGHEST,
                            preferred_element_type=jnp.float32)
    cidxT = cidxT + jnp.where(slotT >= cntC, float(sent), 0.0)
    cidx_ref[0] = jnp.floor(cidxT + 0.5).astype(jnp.int32)
    cavalT = lax.dot_general(aT.astype(jnp.float32), selC,
                             (((1,), (1,)), ((), ())),
                             precision=lax.Precision.HIGHEST,
                             preferred_element_type=jnp.float32)
    caval_ref[0] = jnp.floor(cavalT + 0.5).astype(jnp.int32)


def _sc_gather(idx_hbm, cnt_hbm, emb_hbm, gat_out,
               idxt_v, cnt_v, rows_v, sem, *, rcap, gtile, ncores, rper):
    wid = lax.axis_index("s") * ncores + lax.axis_index("c")
    for rr in range(rper):
        region = wid * rper + rr
        rbase = region * rcap
        pltpu.sync_copy(cnt_hbm.at[region], cnt_v)
        tot = jnp.max(cnt_v[...])
        ntiles = lax.div(tot + gtile - 1, gtile)

        def gbody(t, _):
            @pl.when(t < ntiles)
            def _():
                pltpu.sync_copy(idx_hbm.at[pl.ds(rbase + t * gtile, gtile)],
                                idxt_v)
                pltpu.async_copy(emb_hbm.at[idxt_v], rows_v, sem).wait()
                pltpu.sync_copy(rows_v,
                                gat_out.at[pl.ds(rbase + t * gtile, gtile)])
            return 0

        lax.fori_loop(0, rcap // gtile, gbody, 0, unroll=False)


def _res_mlp_kernel(cnt_ref, gat_ref, embb_ref, wres1_ref, bres1_ref,
                    wres2_ref, bres2_ref, outc_ref, x_v, o_v, sem_in, sem_out,
                    *, rcap, tb):
    r = pl.program_id(0)
    cnt = cnt_ref[r]
    ntiles = (cnt + tb - 1) // tb

    def body(t, _):
        start = r * rcap + t * tb
        cp_in = pltpu.make_async_copy(gat_ref.at[pl.ds(start, tb)], x_v,
                                      sem_in)
        cp_in.start()
        cp_in.wait()
        h = x_v[...] + embb_ref[...]
        z = _elu(_bdot(h, wres1_ref[...]) + bres1_ref[...])
        z = _elu(_bdot(z, wres2_ref[...]) + bres2_ref[...])
        o_v[...] = _elu(h + z)
        cp_out = pltpu.make_async_copy(o_v, outc_ref.at[pl.ds(start, tb)],
                                       sem_out)
        cp_out.start()
        cp_out.wait()
        return 0

    lax.fori_loop(0, ntiles, body, 0, unroll=False)


def _dtable_kernel(bops_ref, wres1_ref, bres1_ref, wres2_ref, bres2_ref,
                   out_ref):
    c = _elu(bops_ref[...])
    z = _elu(jnp.dot(c, wres1_ref[...],
                     preferred_element_type=jnp.float32) + bres1_ref[...])
    z = _elu(jnp.dot(z, wres2_ref[...],
                     preferred_element_type=jnp.float32) + bres2_ref[...])
    out_ref[...] = _elu(c + z)


def _sc_write(a_hbm, dtab_hbm, outc_hbm, idx_hbm, cnt_hbm, out_hbm,
              at_v, idxt_v, cnt_v, rows_v, rows2_v, sem, sem2,
              *, cap, rcap, gtile, ncores, rper):
    wid = lax.axis_index("s") * ncores + lax.axis_index("c")
    base = wid * cap

    def abody(t, _):
        pltpu.sync_copy(a_hbm.at[pl.ds(base + t * gtile, gtile)], at_v)
        pltpu.async_copy(dtab_hbm.at[at_v], rows_v, sem).wait()
        pltpu.sync_copy(rows_v, out_hbm.at[pl.ds(base + t * gtile, gtile)])
        return 0

    lax.fori_loop(0, cap // gtile, abody, 0, unroll=False)

    for rr in range(rper):
        region = wid * rper + rr
        rbase = region * rcap
        pltpu.sync_copy(cnt_hbm.at[region], cnt_v)
        tot = jnp.max(cnt_v[...])
        ntB = lax.div(tot + gtile - 1, gtile)

        def bbody(t, _):
            @pl.when(t < ntB)
            def _():
                pltpu.sync_copy(outc_hbm.at[pl.ds(rbase + t * gtile, gtile)],
                                rows2_v)
                pltpu.sync_copy(idx_hbm.at[pl.ds(rbase + t * gtile, gtile)],
                                idxt_v)
                pltpu.async_copy(rows2_v, out_hbm.at[idxt_v], sem2).wait()
            return 0

        lax.fori_loop(0, rcap // gtile, bbody, 0, unroll=False)


def kernel(x_attr, node_assign, W_pre, b_pre, emb_W, emb_b, W_ops, b_ops,
           W_res1, b_res1, W_res2, b_res2):
    n_attr, d_in = x_attr.shape
    n_total = node_assign.shape[0]
    n_unattr = n_total - n_attr
    n_ops, d_hid, _ = W_ops.shape
    d_mid = W_res1.shape[1]

    assign = node_assign.astype(jnp.int32)

    info = plsc.get_sparse_core_info()
    ncores = info.num_cores
    nw = ncores * info.num_subcores

    B = 512
    BC = 640
    GT = 128
    RPER = 2
    pa = pl.cdiv(n_attr, B) * B
    pu = pl.cdiv(n_unattr, nw * BC * RPER) * nw * BC * RPER
    cap = pu // nw
    nregions = pu // BC
    sent = pu

    b_pre2 = b_pre.reshape(1, d_hid)
    emb_b2 = emb_b.reshape(1, d_hid)
    b_res1_2 = b_res1.reshape(1, d_mid)
    b_res2_2 = b_res2.reshape(1, d_hid)

    full = lambda shape: pl.BlockSpec(shape, lambda *_: (0,) * len(shape))

    x_p = jnp.pad(x_attr, ((0, pa - n_attr), (0, 0)))
    W_all = jnp.transpose(W_ops, (1, 0, 2)).reshape(d_hid, n_ops * d_hid)
    a_attr = jnp.pad(assign[:n_attr], (0, pa - n_attr)).reshape(pa // B, B, 1)
    out_attr = pl.pallas_call(
        functools.partial(_attr_kernel, n_ops=n_ops),
        grid=(pa // B,),
        in_specs=[
            pl.BlockSpec((B, d_in), lambda i: (i, 0)),
            pl.BlockSpec((1, B, 1), lambda i: (i, 0, 0)),
            full((d_in, d_hid)),
            full((1, d_hid)),
            full((d_hid, n_ops * d_hid)),
            full((n_ops, d_hid)),
            full((d_hid, d_mid)),
            full((1, d_mid)),
            full((d_mid, d_hid)),
            full((1, d_hid)),
        ],
        out_specs=pl.BlockSpec((B, d_hid), lambda i: (i, 0)),
        out_shape=jax.ShapeDtypeStruct((pa, d_hid), jnp.float32),
    )(x_p, a_attr, W_pre, b_pre2, W_all, b_ops, W_res1, b_res1_2,
      W_res2, b_res2_2)

    a_un = jnp.pad(assign[n_attr:], (0, pu - n_unattr), constant_values=1)
    aT3 = a_un.reshape(nregions, 1, BC)
    idx3, cidx3, caval3, cnt3 = pl.pallas_call(
        functools.partial(_compact_kernel, bc=BC, sent=sent),
        grid=(nregions,),
        in_specs=[pl.BlockSpec((1, 1, BC), lambda i: (i, 0, 0))],
        out_specs=[pl.BlockSpec((1, 1, BC), lambda i: (i, 0, 0)),
                   pl.BlockSpec((1, 1, BC), lambda i: (i, 0, 0)),
                   pl.BlockSpec((1, 1, BC), lambda i: (i, 0, 0)),
                   pl.BlockSpec((1, 1, 16), lambda i: (i, 0, 0))],
        out_shape=[jax.ShapeDtypeStruct((nregions, 1, BC), jnp.int32),
                   jax.ShapeDtypeStruct((nregions, 1, BC), jnp.int32),
                   jax.ShapeDtypeStruct((nregions, 1, BC), jnp.int32),
                   jax.ShapeDtypeStruct((nregions, 1, 16), jnp.int32)],
    )(aT3)
    idx_arr = idx3.reshape(pu)
    cidx_arr = cidx3.reshape(pu)
    caval_arr = caval3.reshape(pu)
    cnts = cnt3.reshape(nregions, 16)

    emb_p = jnp.pad(emb_W, ((0, pu + 8 - n_unattr), (0, 0)))
    mesh = plsc.VectorSubcoreMesh(core_axis_name="c", subcore_axis_name="s")
    s1 = pl.kernel(
        functools.partial(_sc_gather, rcap=BC, gtile=GT, ncores=ncores,
                          rper=RPER),
        out_type=jax.ShapeDtypeStruct((pu, d_hid), jnp.float32),
        mesh=mesh,
        compiler_params=pltpu.CompilerParams(needs_layout_passes=False),
        scratch_types=[
            pltpu.VMEM((GT,), jnp.int32),
            pltpu.VMEM((16,), jnp.int32),
            pltpu.VMEM((GT, d_hid), jnp.float32),
            pltpu.SemaphoreType.DMA,
        ],
    )
    gat = s1(idx_arr, cnts, emb_p)

    out_c = pl.pallas_call(
        functools.partial(_res_mlp_kernel, rcap=BC, tb=GT),
        grid_spec=pltpu.PrefetchScalarGridSpec(
            num_scalar_prefetch=1,
            grid=(nregions,),
            in_specs=[
                pl.BlockSpec(memory_space=pl.MemorySpace.ANY),
                full((1, d_hid)),
                full((d_hid, d_mid)),
                full((1, d_mid)),
                full((d_mid, d_hid)),
                full((1, d_hid)),
            ],
            out_specs=pl.BlockSpec(memory_space=pl.MemorySpace.ANY),
            scratch_shapes=[
                pltpu.VMEM((GT, d_hid), jnp.float32),
                pltpu.VMEM((GT, d_hid), jnp.float32),
                pltpu.SemaphoreType.DMA,
                pltpu.SemaphoreType.DMA,
            ],
        ),
        out_shape=jax.ShapeDtypeStruct((pu, d_hid), jnp.float32),
    )(cnts[:, 0], gat, emb_b2, W_res1, b_res1_2, W_res2, b_res2_2)

    bops_p = jnp.pad(b_ops, ((1, 0), (0, 0)))
    dtab = pl.pallas_call(
        _dtable_kernel,
        in_specs=[full((n_ops + 1, d_hid)), full((d_hid, d_mid)),
                  full((1, d_mid)), full((d_mid, d_hid)), full((1, d_hid))],
        out_specs=full((n_ops + 1, d_hid)),
        out_shape=jax.ShapeDtypeStruct((n_ops + 1, d_hid), jnp.float32),
    )(bops_p, W_res1, b_res1_2, W_res2, b_res2_2)

    s3 = pl.kernel(
        functools.partial(_sc_write, cap=cap, rcap=BC, gtile=GT,
                          ncores=ncores, rper=RPER),
        out_type=jax.ShapeDtypeStruct((pu + 8, d_hid), jnp.float32),
        mesh=mesh,
        compiler_params=pltpu.CompilerParams(needs_layout_passes=False),
        scratch_types=[
            pltpu.VMEM((GT,), jnp.int32),
            pltpu.VMEM((GT,), jnp.int32),
            pltpu.VMEM((16,), jnp.int32),
            pltpu.VMEM((GT, d_hid), jnp.float32),
            pltpu.VMEM((GT, d_hid), jnp.float32),
            pltpu.SemaphoreType.DMA,
            pltpu.SemaphoreType.DMA,
        ],
    )
    out_un = s3(a_un, dtab, out_c, idx_arr, cnts)

    return jnp.concatenate([out_attr[:n_attr], out_un[:n_unattr]], axis=0)


# speedup vs baseline: 4.5155x; 3.9311x over previous
import functools

import jax
import jax.numpy as jnp
from jax.experimental import pallas as pl


def _elu(x):
    return jnp.where(x > 0, x, jnp.exp(x) - 1.0)


def _bdot(a, b):
    return jnp.dot(a.astype(jnp.bfloat16), b.astype(jnp.bfloat16),
                   preferred_element_type=jnp.float32)


def _attr_kernel(x_ref, a_ref, wpre_ref, bpre_ref, wall_ref, bops_ref,
                 wres1_ref, bres1_ref, wres2_ref, bres2_ref, out_ref, *, n_ops):
    h = _bdot(x_ref[...], wpre_ref[...]) + bpre_ref[...]
    a = a_ref[0]
    d = h.shape[1]
    big = _bdot(h, wall_ref[...])
    ks = 1 + jax.lax.broadcasted_iota(jnp.int32, (1, n_ops), 1)
    oh = (a == ks).astype(jnp.float32)
    acc = jnp.dot(oh, bops_ref[...], preferred_element_type=jnp.float32)
    for k in range(1, n_ops + 1):
        acc = acc + jnp.where(a == k, big[:, (k - 1) * d:k * d], 0.0)
    acc = _elu(acc)
    acc = jnp.where(a == 0, 0.0, acc)
    r = _elu(_bdot(acc, wres1_ref[...]) + bres1_ref[...])
    r = _elu(_bdot(r, wres2_ref[...]) + bres2_ref[...])
    out_ref[...] = _elu(acc + r) + h


def _unattr_kernel(e_ref, a_ref, embb_ref, bops_ref,
                   wres1_ref, bres1_ref, wres2_ref, bres2_ref, out_ref, *, n_ops):
    a = a_ref[0]
    tbl = _elu(bops_ref[...])
    ks = 1 + jax.lax.broadcasted_iota(jnp.int32, (1, n_ops), 1)
    oh = (a == ks).astype(jnp.float32)
    const_part = jnp.dot(oh, tbl, preferred_element_type=jnp.float32)
    emb_part = jnp.where(a == 0, e_ref[...] + embb_ref[...], 0.0)
    h_att = emb_part + const_part
    r = _elu(_bdot(h_att, wres1_ref[...]) + bres1_ref[...])
    r = _elu(_bdot(r, wres2_ref[...]) + bres2_ref[...])
    out_ref[...] = _elu(h_att + r)


def kernel(x_attr, node_assign, W_pre, b_pre, emb_W, emb_b, W_ops, b_ops,
           W_res1, b_res1, W_res2, b_res2):
    n_attr, d_in = x_attr.shape
    n_total = node_assign.shape[0]
    n_unattr = n_total - n_attr
    n_ops, d_hid, _ = W_ops.shape
    d_mid = W_res1.shape[1]

    assign = node_assign.astype(jnp.int32)

    B = 512
    pa = pl.cdiv(n_attr, B) * B
    pu = pl.cdiv(n_unattr, B) * B

    x_p = jnp.pad(x_attr, ((0, pa - n_attr), (0, 0)))
    W_all = jnp.transpose(W_ops, (1, 0, 2)).reshape(d_hid, n_ops * d_hid)
    a_attr = jnp.pad(assign[:n_attr], (0, pa - n_attr)).reshape(pa // B, B, 1)
    e_p = jnp.pad(emb_W, ((0, pu - n_unattr), (0, 0)))
    a_un = jnp.pad(assign[n_attr:], (0, pu - n_unattr)).reshape(pu // B, B, 1)

    b_pre2 = b_pre.reshape(1, d_hid)
    emb_b2 = emb_b.reshape(1, d_hid)
    b_res1_2 = b_res1.reshape(1, d_mid)
    b_res2_2 = b_res2.reshape(1, d_hid)

    full = lambda shape: pl.BlockSpec(shape, lambda *_: (0,) * len(shape))

    out_attr = pl.pallas_call(
        functools.partial(_attr_kernel, n_ops=n_ops),
        grid=(pa // B,),
        in_specs=[
            pl.BlockSpec((B, d_in), lambda i: (i, 0)),
            pl.BlockSpec((1, B, 1), lambda i: (i, 0, 0)),
            full((d_in, d_hid)),
            full((1, d_hid)),
            full((d_hid, n_ops * d_hid)),
            full((n_ops, d_hid)),
            full((d_hid, d_mid)),
            full((1, d_mid)),
            full((d_mid, d_hid)),
            full((1, d_hid)),
        ],
        out_specs=pl.BlockSpec((B, d_hid), lambda i: (i, 0)),
        out_shape=jax.ShapeDtypeStruct((pa, d_hid), jnp.float32),
    )(x_p, a_attr, W_pre, b_pre2, W_all, b_ops, W_res1, b_res1_2,
      W_res2, b_res2_2)

    out_unattr = pl.pallas_call(
        functools.partial(_unattr_kernel, n_ops=n_ops),
        grid=(pu // B,),
        in_specs=[
            pl.BlockSpec((B, d_hid), lambda i: (i, 0)),
            pl.BlockSpec((1, B, 1), lambda i: (i, 0, 0)),
            full((1, d_hid)),
            full((n_ops, d_hid)),
            full((d_hid, d_mid)),
            full((1, d_mid)),
            full((d_mid, d_hid)),
            full((1, d_hid)),
        ],
        out_specs=pl.BlockSpec((B, d_hid), lambda i: (i, 0)),
        out_shape=jax.ShapeDtypeStruct((pu, d_hid), jnp.float32),
    )(e_p, a_un, emb_b2, b_ops, W_res1, b_res1_2, W_res2, b_res2_2)

    return jnp.concatenate([out_attr[:n_attr], out_unattr[:n_unattr]], axis=0)


# speedup vs baseline: 5.0936x; 1.1280x over previous
import functools

import jax
import jax.numpy as jnp
from jax.experimental import pallas as pl


def _elu(x):
    return jnp.where(x > 0, x, jnp.exp(x) - 1.0)


def _bdot(a, b):
    return jnp.dot(a.astype(jnp.bfloat16), b.astype(jnp.bfloat16),
                   preferred_element_type=jnp.float32)


def _attr_kernel(x_ref, a_ref, wpre_ref, bpre_ref, wall_ref, bops_ref,
                 wres1_ref, bres1_ref, wres2_ref, bres2_ref, out_ref, *, n_ops):
    h = _bdot(x_ref[...], wpre_ref[...]) + bpre_ref[...]
    a = a_ref[0]
    d = h.shape[1]
    big = _bdot(h, wall_ref[...])
    ks = 1 + jax.lax.broadcasted_iota(jnp.int32, (1, n_ops), 1)
    oh = (a == ks).astype(jnp.float32)
    acc = jnp.dot(oh, bops_ref[...], preferred_element_type=jnp.float32)
    for k in range(1, n_ops + 1):
        acc = acc + jnp.where(a == k, big[:, (k - 1) * d:k * d], 0.0)
    acc = _elu(acc)
    acc = jnp.where(a == 0, 0.0, acc)
    r = _elu(_bdot(acc, wres1_ref[...]) + bres1_ref[...])
    r = _elu(_bdot(r, wres2_ref[...]) + bres2_ref[...])
    out_ref[...] = _elu(acc + r) + h


def _unattr_kernel(e_ref, a_ref, embb_ref, bops_ref,
                   wres1_ref, bres1_ref, wres2_ref, bres2_ref, out_ref, *, n_ops):
    a = a_ref[0]
    tbl = _elu(bops_ref[...])
    ks = 1 + jax.lax.broadcasted_iota(jnp.int32, (1, n_ops), 1)
    oh = (a == ks).astype(jnp.float32)
    const_part = jnp.dot(oh, tbl, preferred_element_type=jnp.float32)
    emb_part = jnp.where(a == 0, e_ref[...] + embb_ref[...], 0.0)
    h_att = emb_part + const_part
    r = _elu(_bdot(h_att, wres1_ref[...]) + bres1_ref[...])
    r = _elu(_bdot(r, wres2_ref[...]) + bres2_ref[...])
    out_ref[...] = _elu(h_att + r)


def kernel(x_attr, node_assign, W_pre, b_pre, emb_W, emb_b, W_ops, b_ops,
           W_res1, b_res1, W_res2, b_res2):
    n_attr, d_in = x_attr.shape
    n_total = node_assign.shape[0]
    n_unattr = n_total - n_attr
    n_ops, d_hid, _ = W_ops.shape
    d_mid = W_res1.shape[1]

    assign = node_assign.astype(jnp.int32)

    B = 1024
    pa = pl.cdiv(n_attr, B) * B
    pu = pl.cdiv(n_unattr, B) * B

    x_p = jnp.pad(x_attr, ((0, pa - n_attr), (0, 0)))
    W_all = jnp.transpose(W_ops, (1, 0, 2)).reshape(d_hid, n_ops * d_hid)
    a_attr = jnp.pad(assign[:n_attr], (0, pa - n_attr)).reshape(pa // B, B, 1)
    e_p = jnp.pad(emb_W, ((0, pu - n_unattr), (0, 0)))
    a_un = jnp.pad(assign[n_attr:], (0, pu - n_unattr)).reshape(pu // B, B, 1)

    b_pre2 = b_pre.reshape(1, d_hid)
    emb_b2 = emb_b.reshape(1, d_hid)
    b_res1_2 = b_res1.reshape(1, d_mid)
    b_res2_2 = b_res2.reshape(1, d_hid)

    full = lambda shape: pl.BlockSpec(shape, lambda *_: (0,) * len(shape))

    out_attr = pl.pallas_call(
        functools.partial(_attr_kernel, n_ops=n_ops),
        grid=(pa // B,),
        in_specs=[
            pl.BlockSpec((B, d_in), lambda i: (i, 0)),
            pl.BlockSpec((1, B, 1), lambda i: (i, 0, 0)),
            full((d_in, d_hid)),
            full((1, d_hid)),
            full((d_hid, n_ops * d_hid)),
            full((n_ops, d_hid)),
            full((d_hid, d_mid)),
            full((1, d_mid)),
            full((d_mid, d_hid)),
            full((1, d_hid)),
        ],
        out_specs=pl.BlockSpec((B, d_hid), lambda i: (i, 0)),
        out_shape=jax.ShapeDtypeStruct((pa, d_hid), jnp.float32),
    )(x_p, a_attr, W_pre, b_pre2, W_all, b_ops, W_res1, b_res1_2,
      W_res2, b_res2_2)

    out_unattr = pl.pallas_call(
        functools.partial(_unattr_kernel, n_ops=n_ops),
        grid=(pu // B,),
        in_specs=[
            pl.BlockSpec((B, d_hid), lambda i: (i, 0)),
            pl.BlockSpec((1, B, 1), lambda i: (i, 0, 0)),
            full((1, d_hid)),
            full((n_ops, d_hid)),
            full((d_hid, d_mid)),
            full((1, d_mid)),
            full((d_mid, d_hid)),
            full((1, d_hid)),
        ],
        out_specs=pl.BlockSpec((B, d_hid), lambda i: (i, 0)),
        out_shape=jax.ShapeDtypeStruct((pu, d_hid), jnp.float32),
    )(e_p, a_un, emb_b2, b_ops, W_res1, b_res1_2, W_res2, b_res2_2)

    return jnp.concatenate([out_attr[:n_attr], out_unattr[:n_unattr]], axis=0)


# speedup vs baseline: 5.3660x; 1.0535x over previous
import functools

import jax
import jax.numpy as jnp
from jax.experimental import pallas as pl


def _elu(x):
    return jnp.where(x > 0, x, jnp.exp(x) - 1.0)


def _bdot(a, b):
    return jnp.dot(a.astype(jnp.bfloat16), b.astype(jnp.bfloat16),
                   preferred_element_type=jnp.float32)


def _attr_kernel(x_ref, a_ref, wpre_ref, bpre_ref, wall_ref, bops_ref,
                 wres1_ref, bres1_ref, wres2_ref, bres2_ref, out_ref, *, n_ops):
    h = _bdot(x_ref[...], wpre_ref[...]) + bpre_ref[...]
    a = a_ref[0]
    d = h.shape[1]
    big = _bdot(h, wall_ref[...])
    ks = 1 + jax.lax.broadcasted_iota(jnp.int32, (1, n_ops), 1)
    oh = (a == ks).astype(jnp.float32)
    acc = jnp.dot(oh, bops_ref[...], preferred_element_type=jnp.float32)
    for k in range(1, n_ops + 1):
        acc = acc + jnp.where(a == k, big[:, (k - 1) * d:k * d], 0.0)
    acc = _elu(acc)
    acc = jnp.where(a == 0, 0.0, acc)
    r = _elu(_bdot(acc, wres1_ref[...]) + bres1_ref[...])
    r = _elu(_bdot(r, wres2_ref[...]) + bres2_ref[...])
    out_ref[...] = _elu(acc + r) + h


def _unattr_kernel(e_ref, a_ref, embb_ref, bops_ref,
                   wres1_ref, bres1_ref, wres2_ref, bres2_ref, out_ref, *, n_ops):
    a = a_ref[0]
    tbl = _elu(bops_ref[...])
    ks = 1 + jax.lax.broadcasted_iota(jnp.int32, (1, n_ops), 1)
    oh = (a == ks).astype(jnp.float32)
    const_part = jnp.dot(oh, tbl, preferred_element_type=jnp.float32)
    emb_part = jnp.where(a == 0, e_ref[...] + embb_ref[...], 0.0)
    h_att = emb_part + const_part
    r = _elu(_bdot(h_att, wres1_ref[...]) + bres1_ref[...])
    r = _elu(_bdot(r, wres2_ref[...]) + bres2_ref[...])
    out_ref[...] = _elu(h_att + r)


def kernel(x_attr, node_assign, W_pre, b_pre, emb_W, emb_b, W_ops, b_ops,
           W_res1, b_res1, W_res2, b_res2):
    n_attr, d_in = x_attr.shape
    n_total = node_assign.shape[0]
    n_unattr = n_total - n_attr
    n_ops, d_hid, _ = W_ops.shape
    d_mid = W_res1.shape[1]

    assign = node_assign.astype(jnp.int32)

    B = 2048
    pa = pl.cdiv(n_attr, B) * B
    pu = pl.cdiv(n_unattr, B) * B

    x_p = jnp.pad(x_attr, ((0, pa - n_attr), (0, 0)))
    W_all = jnp.transpose(W_ops, (1, 0, 2)).reshape(d_hid, n_ops * d_hid)
    a_attr = jnp.pad(assign[:n_attr], (0, pa - n_attr)).reshape(pa // B, B, 1)
    e_p = jnp.pad(emb_W, ((0, pu - n_unattr), (0, 0)))
    a_un = jnp.pad(assign[n_attr:], (0, pu - n_unattr)).reshape(pu // B, B, 1)

    b_pre2 = b_pre.reshape(1, d_hid)
    emb_b2 = emb_b.reshape(1, d_hid)
    b_res1_2 = b_res1.reshape(1, d_mid)
    b_res2_2 = b_res2.reshape(1, d_hid)

    full = lambda shape: pl.BlockSpec(shape, lambda *_: (0,) * len(shape))

    out_attr = pl.pallas_call(
        functools.partial(_attr_kernel, n_ops=n_ops),
        grid=(pa // B,),
        in_specs=[
            pl.BlockSpec((B, d_in), lambda i: (i, 0)),
            pl.BlockSpec((1, B, 1), lambda i: (i, 0, 0)),
            full((d_in, d_hid)),
            full((1, d_hid)),
            full((d_hid, n_ops * d_hid)),
            full((n_ops, d_hid)),
            full((d_hid, d_mid)),
            full((1, d_mid)),
            full((d_mid, d_hid)),
            full((1, d_hid)),
        ],
        out_specs=pl.BlockSpec((B, d_hid), lambda i: (i, 0)),
        out_shape=jax.ShapeDtypeStruct((pa, d_hid), jnp.float32),
    )(x_p, a_attr, W_pre, b_pre2, W_all, b_ops, W_res1, b_res1_2,
      W_res2, b_res2_2)

    out_unattr = pl.pallas_call(
        functools.partial(_unattr_kernel, n_ops=n_ops),
        grid=(pu // B,),
        in_specs=[
            pl.BlockSpec((B, d_hid), lambda i: (i, 0)),
            pl.BlockSpec((1, B, 1), lambda i: (i, 0, 0)),
            full((1, d_hid)),
            full((n_ops, d_hid)),
            full((d_hid, d_mid)),
            full((1, d_mid)),
            full((d_mid, d_hid)),
            full((1, d_hid)),
        ],
        out_specs=pl.BlockSpec((B, d_hid), lambda i: (i, 0)),
        out_shape=jax.ShapeDtypeStruct((pu, d_hid), jnp.float32),
    )(e_p, a_un, emb_b2, b_ops, W_res1, b_res1_2, W_res2, b_res2_2)

    return jnp.concatenate([out_attr[:n_attr], out_unattr[:n_unattr]], axis=0)
